# Initial kernel scaffold; baseline (speedup 1.0000x reference)
import jax
import jax.numpy as jnp
from jax.experimental import pallas as pl


def kernel(query_features, class_embeddings, class_indices):
    raise NotImplementedError("write your pallas kernel here")


# speedup vs baseline: 2.5243x; 2.5243x over previous
import functools

import jax
import jax.numpy as jnp
from jax import lax
from jax.experimental import pallas as pl
from jax.experimental.pallas import tpu as pltpu
from jax.experimental.pallas import tpu_sc as plsc

B = 65536
D = 256
L = 16
NC, NS = 2, 16
NW = NC * NS
RPW = B // NW
CH = 128
NCHUNK = RPW // CH
LG = D // L


def _sc_partial_sums(q4, table, idx3):
    mesh = plsc.VectorSubcoreMesh(core_axis_name="c", subcore_axis_name="s")

    @functools.partial(
        pl.kernel,
        mesh=mesh,
        out_type=jax.ShapeDtypeStruct((NW, L), jnp.float32),
        scratch_types=[
            pltpu.VMEM((NCHUNK, CH), jnp.int32),
            pltpu.VMEM((CH, D), jnp.float32),
            pltpu.VMEM((CH, D), jnp.float32),
            pltpu.VMEM((L,), jnp.float32),
            pltpu.SemaphoreType.DMA,
        ],
    )
    def k(q_hbm, tab_hbm, idx_hbm, out_hbm, idx_v, rows_v, qv, acc_v, sem):
        wid = lax.axis_index("s") * NC + lax.axis_index("c")
        pltpu.sync_copy(idx_hbm.at[wid], idx_v)

        def chunk_body(g, accs):
            gcp = pltpu.async_copy(tab_hbm.at[idx_v.at[g]], rows_v, sem)
            pltpu.sync_copy(q_hbm.at[wid, g], qv)
            gcp.wait()

            def row_body(i, accs):
                out = []
                for j in range(LG):
                    dlt = rows_v[i, pl.ds(L * j, L)] - qv[i, pl.ds(L * j, L)]
                    out.append(accs[j] + dlt * dlt)
                return tuple(out)

            return lax.fori_loop(0, CH, row_body, accs)

        zero = jnp.zeros((L,), jnp.float32)
        accs = lax.fori_loop(0, NCHUNK, chunk_body, (zero,) * LG)
        total = accs[0]
        for j in range(1, LG):
            total = total + accs[j]
        acc_v[...] = total
        pltpu.sync_copy(acc_v, out_hbm.at[wid])

    return k(q4, table, idx3)


def kernel(query_features, class_embeddings, class_indices):
    q4 = query_features.reshape(NW, NCHUNK, CH, D)
    idx3 = class_indices.astype(jnp.int32).reshape(NW, NCHUNK, CH)
    partial = _sc_partial_sums(q4, class_embeddings, idx3)
    return jnp.sum(partial) * (1.25 / (B * D))


# speedup vs baseline: 3.7213x; 1.4742x over previous
import functools

import jax
import jax.numpy as jnp
from jax import lax
from jax.experimental import pallas as pl
from jax.experimental.pallas import tpu as pltpu
from jax.experimental.pallas import tpu_sc as plsc

B = 65536
D = 256
L = 16
NC, NS = 2, 16
NW = NC * NS
RPW = B // NW
CH = 64
NCHUNK = RPW // CH
LG = D // L


def _sc_partial_sums(q4, table, idx3):
    mesh = plsc.VectorSubcoreMesh(core_axis_name="c", subcore_axis_name="s")

    @functools.partial(
        pl.kernel,
        mesh=mesh,
        out_type=jax.ShapeDtypeStruct((NW, L), jnp.float32),
        scratch_types=[
            pltpu.VMEM((NCHUNK, CH), jnp.int32),
            pltpu.VMEM((CH, D), jnp.float32),
            pltpu.VMEM((CH, D), jnp.float32),
            pltpu.VMEM((CH, D), jnp.float32),
            pltpu.VMEM((CH, D), jnp.float32),
            pltpu.VMEM((L,), jnp.float32),
            pltpu.SemaphoreType.DMA,
            pltpu.SemaphoreType.DMA,
        ],
    )
    def k(q_hbm, tab_hbm, idx_hbm, out_hbm,
          idx_v, rows0, rows1, q0, q1, acc_v, sem0, sem1):
        wid = lax.axis_index("s") * NC + lax.axis_index("c")
        pltpu.sync_copy(idx_hbm.at[wid], idx_v)

        bufs = ((rows0, q0, sem0), (rows1, q1, sem1))

        def start(c, b):
            rows_b, q_b, sem_b = bufs[b]
            pltpu.async_copy(tab_hbm.at[idx_v.at[c]], rows_b, sem_b)
            pltpu.async_copy(q_hbm.at[wid, c], q_b, sem_b)

        def wait_and_compute(c, b, accs):
            rows_b, q_b, sem_b = bufs[b]
            pltpu.make_async_copy(tab_hbm.at[pl.ds(0, CH)], rows_b, sem_b).wait()
            pltpu.make_async_copy(q_hbm.at[wid, c], q_b, sem_b).wait()

            def row_body(i, accs):
                out = []
                for j in range(LG):
                    dlt = rows_b[i, pl.ds(L * j, L)] - q_b[i, pl.ds(L * j, L)]
                    out.append(accs[j] + dlt * dlt)
                return tuple(out)

            return lax.fori_loop(0, CH, row_body, accs)

        start(0, 0)
        start(1, 1)
        zero = jnp.zeros((L,), jnp.float32)

        def outer(i, accs):
            c0 = 2 * i
            accs = wait_and_compute(c0, 0, accs)

            @pl.when(c0 + 2 < NCHUNK)
            def _():
                start(c0 + 2, 0)

            accs = wait_and_compute(c0 + 1, 1, accs)

            @pl.when(c0 + 3 < NCHUNK)
            def _():
                start(c0 + 3, 1)

            return accs

        accs = lax.fori_loop(0, NCHUNK // 2, outer, (zero,) * LG)
        total = accs[0]
        for j in range(1, LG):
            total = total + accs[j]
        acc_v[...] = total
        pltpu.sync_copy(acc_v, out_hbm.at[wid])

    return k(q4, table, idx3)


def kernel(query_features, class_embeddings, class_indices):
    q4 = query_features.reshape(NW, NCHUNK, CH, D)
    idx3 = class_indices.astype(jnp.int32).reshape(NW, NCHUNK, CH)
    partial = _sc_partial_sums(q4, class_embeddings, idx3)
    return jnp.sum(partial) * (1.25 / (B * D))
